# baseline (speedup 1.0000x reference)
import functools

import jax
import jax.numpy as jnp
from jax.experimental import pallas as pl
from jax.experimental.pallas import tpu as pltpu
from jax.experimental.pallas import tpu_sc as plsc

_NC = 2
_NS = 16
_NW = _NC * _NS


def _ymm_body(net, x_ref, w_ref, y_ref):
    xb = x_ref[...]
    for et in range(net):
        y_ref[et] = jax.lax.dot_general(
            xb, w_ref[et], (((1,), (1,)), ((), ())),
            preferred_element_type=jnp.float32)


def _lin_body(x_ref, w_ref, b_ref, z_ref):
    z_ref[...] = jax.lax.dot_general(
        x_ref[...], w_ref[...], (((1,), (1,)), ((), ())),
        preferred_element_type=jnp.float32) + b_ref[...]


def _bnfused_body(n_rows, nb, z_ref, p0_ref, p1_ref, g_ref, b_ref, o_ref,
                  h_ref, s_ref):
    it = pl.program_id(0)
    i = jax.lax.rem(it, nb)

    @pl.when(it < nb)
    def _():
        hb = jnp.maximum(z_ref[...] + p0_ref[0] + p1_ref[0], 0.0)
        h_ref[pl.ds(i * z_ref.shape[0], z_ref.shape[0]), :] = hb
        st = jnp.concatenate(
            [jnp.sum(hb, axis=0, keepdims=True),
             jnp.sum(hb * hb, axis=0, keepdims=True)], axis=0)

        @pl.when(it == 0)
        def _():
            s_ref[...] = st

        @pl.when(it > 0)
        def _():
            s_ref[...] = s_ref[...] + st

    @pl.when(it >= nb)
    def _():
        s = s_ref[...]
        mean = s[0:1] * (1.0 / n_rows)
        var = s[1:2] * (1.0 / n_rows) - mean * mean
        inv = jax.lax.rsqrt(var + 1e-5)
        hb = h_ref[pl.ds(i * o_ref.shape[0], o_ref.shape[0]), :]
        o_ref[...] = (hb - mean) * (g_ref[...] * inv) + b_ref[...]


def kernel(x, edge_index, edge_attr, conv_w, conv_b, lin_w, lin_b, bn_gamma, bn_beta):
    n, d = x.shape
    e = edge_attr.shape[0]
    net = conv_w.shape[0]

    cb = 80
    npad = ((n + _NS * 8 - 1) // (_NS * 8)) * (_NS * 8)
    rpt = npad // _NS
    nchunk = -(-e // (_NW * cb))
    epw = nchunk * cb
    epad = _NW * epw - e

    bn = 1000
    nb = n // bn
    y = pl.pallas_call(
        functools.partial(_ymm_body, net),
        grid=(nb,),
        in_specs=[pl.BlockSpec((bn, d), lambda i: (i, 0)),
                  pl.BlockSpec((net, d, d), lambda i: (0, 0, 0))],
        out_specs=pl.BlockSpec((net, bn, d), lambda i: (0, i, 0)),
        out_shape=jax.ShapeDtypeStruct((net, n, d), jnp.float32),
    )(x, conv_w)
    yf = y.reshape(net * n, d)

    if epad:
        pad0 = jnp.zeros((epad,), jnp.int32)
        src = jnp.concatenate([edge_index[0], pad0]).reshape(_NW, epw)
        etr = jnp.concatenate([edge_attr, pad0]).reshape(_NW, epw)
        dst = jnp.concatenate([edge_index[1], pad0 + n]).reshape(_NW, nchunk, cb)
    else:
        src = edge_index[0].reshape(_NW, epw)
        etr = edge_attr.reshape(_NW, epw)
        dst = edge_index[1].reshape(_NW, nchunk, cb)
    zeros = jnp.zeros((npad, d), jnp.float32)

    mesh = plsc.VectorSubcoreMesh(core_axis_name="c", subcore_axis_name="s",
                                  num_cores=_NC, num_subcores=_NS)

    @functools.partial(
        pl.kernel,
        out_type=jax.ShapeDtypeStruct((_NC, npad, d), jnp.float32),
        mesh=mesh,
        scratch_types=[
            pltpu.VMEM((epw,), jnp.int32),
            pltpu.VMEM((epw,), jnp.int32),
            pltpu.VMEM((nchunk, cb), jnp.int32),
            pltpu.VMEM((cb, d), jnp.float32),
            pltpu.VMEM_SHARED((npad, d), jnp.float32),
            pltpu.SemaphoreType.DMA,
        ],
    )
    def _sc_edge(src_h, et_h, dst_h, y_h, z_h, out_h,
                 gidx, etv, didx, rb0, acc, sg0):
        cid = jax.lax.axis_index("c")
        sid = jax.lax.axis_index("s")
        wid = sid * _NC + cid
        pltpu.sync_copy(src_h.at[wid], gidx)
        pltpu.sync_copy(et_h.at[wid], etv)
        pltpu.sync_copy(dst_h.at[wid], didx)
        pltpu.sync_copy(z_h.at[pl.ds(sid * rpt, rpt)],
                        acc.at[pl.ds(sid * rpt, rpt)])

        def cbody(i, carry):
            sl = pl.ds(i * 16, 16)
            gidx[sl] = etv[sl] * n + gidx[sl]
            return carry

        jax.lax.fori_loop(0, epw // 16, cbody, 0)
        plsc.subcore_barrier()

        def chunk(j, carry):
            pltpu.async_copy(y_h.at[gidx.at[pl.ds(j * cb, cb)]], rb0, sg0).wait()
            pltpu.sync_copy(rb0, acc.at[didx.at[j]], add=True)
            return carry

        jax.lax.fori_loop(0, nchunk, chunk, 0)
        plsc.subcore_barrier()
        pltpu.sync_copy(acc.at[pl.ds(sid * rpt, rpt)],
                        out_h.at[cid, pl.ds(sid * rpt, rpt)])

    partials = _sc_edge(src, etr, dst, yf, zeros)

    btot = (lin_b + jnp.sum(conv_b, axis=0)).reshape(1, d)
    z = pl.pallas_call(
        _lin_body,
        grid=(nb,),
        in_specs=[pl.BlockSpec((bn, d), lambda i: (i, 0)),
                  pl.BlockSpec((d, d), lambda i: (0, 0)),
                  pl.BlockSpec((1, d), lambda i: (0, 0))],
        out_specs=pl.BlockSpec((bn, d), lambda i: (i, 0)),
        out_shape=jax.ShapeDtypeStruct((n, d), jnp.float32),
    )(x, lin_w, btot)

    out = pl.pallas_call(
        functools.partial(_bnfused_body, n, nb),
        grid=(2 * nb,),
        in_specs=[pl.BlockSpec((bn, d), lambda it: (it % nb, 0)),
                  pl.BlockSpec((1, bn, d), lambda it: (0, it % nb, 0)),
                  pl.BlockSpec((1, bn, d), lambda it: (1, it % nb, 0)),
                  pl.BlockSpec((1, d), lambda it: (0, 0)),
                  pl.BlockSpec((1, d), lambda it: (0, 0))],
        out_specs=pl.BlockSpec((bn, d), lambda it: (it % nb, 0)),
        out_shape=jax.ShapeDtypeStruct((n, d), jnp.float32),
        scratch_shapes=[pltpu.VMEM((n, d), jnp.float32),
                        pltpu.VMEM((2, d), jnp.float32)],
    )(z, partials, partials, bn_gamma.reshape(1, d), bn_beta.reshape(1, d))
    return out


# speedup vs baseline: 1.0165x; 1.0165x over previous
import functools

import jax
import jax.numpy as jnp
from jax.experimental import pallas as pl
from jax.experimental.pallas import tpu as pltpu
from jax.experimental.pallas import tpu_sc as plsc

_NC = 2
_NS = 16
_NW = _NC * _NS


def _ymm_body(net, x_ref, w_ref, y_ref):
    xb = x_ref[...]
    for et in range(net):
        y_ref[et] = jax.lax.dot_general(
            xb, w_ref[et], (((1,), (1,)), ((), ())),
            preferred_element_type=jnp.float32)


def _lin_body(x_ref, w_ref, b_ref, z_ref):
    z_ref[...] = jax.lax.dot_general(
        x_ref[...], w_ref[...], (((1,), (1,)), ((), ())),
        preferred_element_type=jnp.float32) + b_ref[...]


def _bnfused_body(n_rows, nb, z_ref, p0_ref, p1_ref, g_ref, b_ref, o_ref,
                  h_ref, s_ref):
    it = pl.program_id(0)
    i = jax.lax.rem(it, nb)

    @pl.when(it < nb)
    def _():
        hb = jnp.maximum(z_ref[...] + p0_ref[0] + p1_ref[0], 0.0)
        h_ref[pl.ds(i * z_ref.shape[0], z_ref.shape[0]), :] = hb
        st = jnp.concatenate(
            [jnp.sum(hb, axis=0, keepdims=True),
             jnp.sum(hb * hb, axis=0, keepdims=True)], axis=0)

        @pl.when(it == 0)
        def _():
            s_ref[...] = st

        @pl.when(it > 0)
        def _():
            s_ref[...] = s_ref[...] + st

    @pl.when(it >= nb)
    def _():
        s = s_ref[...]
        mean = s[0:1] * (1.0 / n_rows)
        var = s[1:2] * (1.0 / n_rows) - mean * mean
        inv = jax.lax.rsqrt(var + 1e-5)
        hb = h_ref[pl.ds(i * o_ref.shape[0], o_ref.shape[0]), :]
        o_ref[...] = (hb - mean) * (g_ref[...] * inv) + b_ref[...]


def kernel(x, edge_index, edge_attr, conv_w, conv_b, lin_w, lin_b, bn_gamma, bn_beta):
    n, d = x.shape
    e = edge_attr.shape[0]
    net = conv_w.shape[0]

    cb = 80
    npad = ((n + _NS * 8 - 1) // (_NS * 8)) * (_NS * 8)
    rpt = npad // _NS
    nchunk = -(-e // (_NW * cb))
    epw = nchunk * cb
    epad = _NW * epw - e

    bn = 1000
    nb = n // bn
    y = pl.pallas_call(
        functools.partial(_ymm_body, net),
        grid=(nb,),
        in_specs=[pl.BlockSpec((bn, d), lambda i: (i, 0)),
                  pl.BlockSpec((net, d, d), lambda i: (0, 0, 0))],
        out_specs=pl.BlockSpec((net, bn, d), lambda i: (0, i, 0)),
        out_shape=jax.ShapeDtypeStruct((net, n, d), jnp.float32),
    )(x, conv_w)
    yf = y.reshape(net * n, d)

    if epad:
        pad0 = jnp.zeros((epad,), jnp.int32)
        src = jnp.concatenate([edge_index[0], pad0]).reshape(_NW, epw)
        etr = jnp.concatenate([edge_attr, pad0]).reshape(_NW, epw)
        dst = jnp.concatenate([edge_index[1], pad0 + n]).reshape(_NW, nchunk, cb)
    else:
        src = edge_index[0].reshape(_NW, epw)
        etr = edge_attr.reshape(_NW, epw)
        dst = edge_index[1].reshape(_NW, nchunk, cb)
    zeros = jnp.zeros((npad, d), jnp.float32)

    mesh = plsc.VectorSubcoreMesh(core_axis_name="c", subcore_axis_name="s",
                                  num_cores=_NC, num_subcores=_NS)

    @functools.partial(
        pl.kernel,
        out_type=jax.ShapeDtypeStruct((_NC, npad, d), jnp.float32),
        mesh=mesh,
        scratch_types=[
            pltpu.VMEM((epw,), jnp.int32),
            pltpu.VMEM((epw,), jnp.int32),
            pltpu.VMEM((nchunk, cb), jnp.int32),
            pltpu.VMEM((cb, d), jnp.float32),
            pltpu.VMEM_SHARED((npad, d), jnp.float32),
            pltpu.SemaphoreType.DMA,
        ],
    )
    def _sc_edge(src_h, et_h, dst_h, y_h, z_h, out_h,
                 gidx, etv, didx, rb0, acc, sg0):
        cid = jax.lax.axis_index("c")
        sid = jax.lax.axis_index("s")
        wid = sid * _NC + cid
        pltpu.sync_copy(src_h.at[wid], gidx)
        pltpu.sync_copy(et_h.at[wid], etv)
        pltpu.sync_copy(dst_h.at[wid], didx)
        pltpu.sync_copy(z_h.at[pl.ds(sid * rpt, rpt)],
                        acc.at[pl.ds(sid * rpt, rpt)])

        def cbody(i, carry):
            sl = pl.ds(i * 16, 16)
            gidx[sl] = etv[sl] * n + gidx[sl]
            return carry

        jax.lax.fori_loop(0, epw // 16, cbody, 0)
        plsc.subcore_barrier()

        def chunk(j, carry):
            pltpu.async_copy(y_h.at[gidx.at[pl.ds(j * cb, cb)]], rb0, sg0).wait()
            pltpu.sync_copy(rb0, acc.at[didx.at[j]], add=True)
            return carry

        jax.lax.fori_loop(0, nchunk, chunk, 0)
        plsc.subcore_barrier()
        pltpu.sync_copy(acc.at[pl.ds(sid * rpt, rpt)],
                        out_h.at[cid, pl.ds(sid * rpt, rpt)])

    partials = _sc_edge(src, etr, dst, yf, zeros)

    btot = (lin_b + jnp.sum(conv_b, axis=0)).reshape(1, d)
    z = pl.pallas_call(
        _lin_body,
        grid=(nb,),
        in_specs=[pl.BlockSpec((bn, d), lambda i: (i, 0)),
                  pl.BlockSpec((d, d), lambda i: (0, 0)),
                  pl.BlockSpec((1, d), lambda i: (0, 0))],
        out_specs=pl.BlockSpec((bn, d), lambda i: (i, 0)),
        out_shape=jax.ShapeDtypeStruct((n, d), jnp.float32),
    )(x, lin_w, btot)

    out = pl.pallas_call(
        functools.partial(_bnfused_body, n, nb),
        grid=(2 * nb,),
        in_specs=[pl.BlockSpec((bn, d), lambda it: (jnp.where(it < nb, it, 0), 0)),
                  pl.BlockSpec((1, bn, d),
                               lambda it: (0, jnp.where(it < nb, it, 0), 0)),
                  pl.BlockSpec((1, bn, d),
                               lambda it: (1, jnp.where(it < nb, it, 0), 0)),
                  pl.BlockSpec((1, d), lambda it: (0, 0)),
                  pl.BlockSpec((1, d), lambda it: (0, 0))],
        out_specs=pl.BlockSpec((bn, d), lambda it: (it % nb, 0)),
        out_shape=jax.ShapeDtypeStruct((n, d), jnp.float32),
        scratch_shapes=[pltpu.VMEM((n, d), jnp.float32),
                        pltpu.VMEM((2, d), jnp.float32)],
    )(z, partials, partials, bn_gamma.reshape(1, d), bn_beta.reshape(1, d))
    return out


# speedup vs baseline: 1.0313x; 1.0146x over previous
import functools

import jax
import jax.numpy as jnp
from jax.experimental import pallas as pl
from jax.experimental.pallas import tpu as pltpu
from jax.experimental.pallas import tpu_sc as plsc

_NC = 2
_NS = 16
_NW = _NC * _NS


def _ymm_body(net, x_ref, w_ref, y_ref):
    xb = x_ref[...]
    for et in range(net):
        y_ref[et] = jax.lax.dot_general(
            xb, w_ref[et], (((1,), (1,)), ((), ())),
            preferred_element_type=jnp.float32)


def _lin_body(x_ref, w_ref, b_ref, z_ref):
    z_ref[...] = jax.lax.dot_general(
        x_ref[...], w_ref[...], (((1,), (1,)), ((), ())),
        preferred_element_type=jnp.float32) + b_ref[...]


def _bnfused_body(n_rows, nb, z_ref, p0_ref, p1_ref, g_ref, b_ref, o_ref,
                  h_ref, s_ref):
    it = pl.program_id(0)
    i = jax.lax.rem(it, nb)

    @pl.when(it < nb)
    def _():
        hb = jnp.maximum(z_ref[...] + p0_ref[0] + p1_ref[0], 0.0)
        h_ref[pl.ds(i * z_ref.shape[0], z_ref.shape[0]), :] = hb
        st = jnp.concatenate(
            [jnp.sum(hb, axis=0, keepdims=True),
             jnp.sum(hb * hb, axis=0, keepdims=True)], axis=0)

        @pl.when(it == 0)
        def _():
            s_ref[...] = st

        @pl.when(it > 0)
        def _():
            s_ref[...] = s_ref[...] + st

    @pl.when(it >= nb)
    def _():
        s = s_ref[...]
        mean = s[0:1] * (1.0 / n_rows)
        var = s[1:2] * (1.0 / n_rows) - mean * mean
        inv = jax.lax.rsqrt(var + 1e-5)
        hb = h_ref[pl.ds(i * o_ref.shape[0], o_ref.shape[0]), :]
        o_ref[...] = (hb - mean) * (g_ref[...] * inv) + b_ref[...]


def kernel(x, edge_index, edge_attr, conv_w, conv_b, lin_w, lin_b, bn_gamma, bn_beta):
    n, d = x.shape
    e = edge_attr.shape[0]
    net = conv_w.shape[0]

    cb = 80
    npad = ((n + _NS * 8 - 1) // (_NS * 8)) * (_NS * 8)
    rpt = npad // _NS
    nchunk = -(-e // (_NW * cb))
    epw = nchunk * cb
    epad = _NW * epw - e

    bn = 1000
    nb = n // bn
    y = pl.pallas_call(
        functools.partial(_ymm_body, net),
        grid=(nb,),
        in_specs=[pl.BlockSpec((bn, d), lambda i: (i, 0)),
                  pl.BlockSpec((net, d, d), lambda i: (0, 0, 0))],
        out_specs=pl.BlockSpec((net, bn, d), lambda i: (0, i, 0)),
        out_shape=jax.ShapeDtypeStruct((net, n, d), jnp.float32),
    )(x, conv_w)
    yf = y.reshape(net * n, d)

    gix = edge_attr * n + edge_index[0]
    if epad:
        pad0 = jnp.zeros((epad,), jnp.int32)
        gix = jnp.concatenate([gix, pad0]).reshape(_NW, nchunk, cb)
        dst = jnp.concatenate([edge_index[1], pad0 + n]).reshape(_NW, nchunk, cb)
    else:
        gix = gix.reshape(_NW, nchunk, cb)
        dst = edge_index[1].reshape(_NW, nchunk, cb)
    zeros = jnp.zeros((npad, d), jnp.float32)

    mesh = plsc.VectorSubcoreMesh(core_axis_name="c", subcore_axis_name="s",
                                  num_cores=_NC, num_subcores=_NS)

    @functools.partial(
        pl.kernel,
        out_type=jax.ShapeDtypeStruct((_NC, npad, d), jnp.float32),
        mesh=mesh,
        scratch_types=[
            pltpu.VMEM((nchunk, cb), jnp.int32),
            pltpu.VMEM((nchunk, cb), jnp.int32),
            pltpu.VMEM((cb, d), jnp.float32),
            pltpu.VMEM_SHARED((npad, d), jnp.float32),
            pltpu.SemaphoreType.DMA,
        ],
    )
    def _sc_edge(gix_h, dst_h, y_h, z_h, out_h,
                 gidx, didx, rb0, acc, sg0):
        cid = jax.lax.axis_index("c")
        sid = jax.lax.axis_index("s")
        wid = sid * _NC + cid
        pltpu.sync_copy(gix_h.at[wid], gidx)
        pltpu.sync_copy(dst_h.at[wid], didx)
        pltpu.sync_copy(z_h.at[pl.ds(sid * rpt, rpt)],
                        acc.at[pl.ds(sid * rpt, rpt)])
        plsc.subcore_barrier()

        def chunk(j, carry):
            pltpu.async_copy(y_h.at[gidx.at[j]], rb0, sg0).wait()
            pltpu.sync_copy(rb0, acc.at[didx.at[j]], add=True)
            return carry

        jax.lax.fori_loop(0, nchunk, chunk, 0)
        plsc.subcore_barrier()
        pltpu.sync_copy(acc.at[pl.ds(sid * rpt, rpt)],
                        out_h.at[cid, pl.ds(sid * rpt, rpt)])

    partials = _sc_edge(gix, dst, yf, zeros)

    btot = (lin_b + jnp.sum(conv_b, axis=0)).reshape(1, d)
    z = pl.pallas_call(
        _lin_body,
        grid=(nb,),
        in_specs=[pl.BlockSpec((bn, d), lambda i: (i, 0)),
                  pl.BlockSpec((d, d), lambda i: (0, 0)),
                  pl.BlockSpec((1, d), lambda i: (0, 0))],
        out_specs=pl.BlockSpec((bn, d), lambda i: (i, 0)),
        out_shape=jax.ShapeDtypeStruct((n, d), jnp.float32),
    )(x, lin_w, btot)

    out = pl.pallas_call(
        functools.partial(_bnfused_body, n, nb),
        grid=(2 * nb,),
        in_specs=[pl.BlockSpec((bn, d), lambda it: (jnp.where(it < nb, it, 0), 0)),
                  pl.BlockSpec((1, bn, d),
                               lambda it: (0, jnp.where(it < nb, it, 0), 0)),
                  pl.BlockSpec((1, bn, d),
                               lambda it: (1, jnp.where(it < nb, it, 0), 0)),
                  pl.BlockSpec((1, d), lambda it: (0, 0)),
                  pl.BlockSpec((1, d), lambda it: (0, 0))],
        out_specs=pl.BlockSpec((bn, d), lambda it: (it % nb, 0)),
        out_shape=jax.ShapeDtypeStruct((n, d), jnp.float32),
        scratch_shapes=[pltpu.VMEM((n, d), jnp.float32),
                        pltpu.VMEM((2, d), jnp.float32)],
    )(z, partials, partials, bn_gamma.reshape(1, d), bn_beta.reshape(1, d))
    return out


# speedup vs baseline: 1.0577x; 1.0255x over previous
import functools

import jax
import jax.numpy as jnp
from jax.experimental import pallas as pl
from jax.experimental.pallas import tpu as pltpu
from jax.experimental.pallas import tpu_sc as plsc

_NC = 2
_NS = 16
_NW = _NC * _NS


def _ymm_body(net, x_ref, w_ref, y_ref):
    xb = x_ref[...]
    for et in range(net):
        y_ref[et] = jax.lax.dot_general(
            xb, w_ref[et], (((1,), (1,)), ((), ())),
            preferred_element_type=jnp.float32)


def _lin_body(x_ref, w_ref, b_ref, z_ref):
    z_ref[...] = jax.lax.dot_general(
        x_ref[...], w_ref[...], (((1,), (1,)), ((), ())),
        preferred_element_type=jnp.float32) + b_ref[...]


def _bnfused_body(n_rows, nb, z_ref, p0_ref, p1_ref, g_ref, b_ref, o_ref,
                  h_ref, s_ref):
    it = pl.program_id(0)
    i = jax.lax.rem(it, nb)

    @pl.when(it < nb)
    def _():
        hb = jnp.maximum(z_ref[...] + p0_ref[0] + p1_ref[0], 0.0)
        h_ref[pl.ds(i * z_ref.shape[0], z_ref.shape[0]), :] = hb
        st = jnp.concatenate(
            [jnp.sum(hb, axis=0, keepdims=True),
             jnp.sum(hb * hb, axis=0, keepdims=True)], axis=0)

        @pl.when(it == 0)
        def _():
            s_ref[...] = st

        @pl.when(it > 0)
        def _():
            s_ref[...] = s_ref[...] + st

    @pl.when(it >= nb)
    def _():
        s = s_ref[...]
        mean = s[0:1] * (1.0 / n_rows)
        var = s[1:2] * (1.0 / n_rows) - mean * mean
        inv = jax.lax.rsqrt(var + 1e-5)
        hb = h_ref[pl.ds(i * o_ref.shape[0], o_ref.shape[0]), :]
        o_ref[...] = (hb - mean) * (g_ref[...] * inv) + b_ref[...]


def kernel(x, edge_index, edge_attr, conv_w, conv_b, lin_w, lin_b, bn_gamma, bn_beta):
    n, d = x.shape
    e = edge_attr.shape[0]
    net = conv_w.shape[0]

    cb = 80
    npad = ((n + _NS * 8 - 1) // (_NS * 8)) * (_NS * 8)
    rpt = npad // _NS
    nchunk = -(-e // (_NW * cb))
    epw = nchunk * cb
    epad = _NW * epw - e

    bn = 2000
    nb = n // bn
    y = pl.pallas_call(
        functools.partial(_ymm_body, net),
        grid=(nb,),
        in_specs=[pl.BlockSpec((bn, d), lambda i: (i, 0)),
                  pl.BlockSpec((net, d, d), lambda i: (0, 0, 0))],
        out_specs=pl.BlockSpec((net, bn, d), lambda i: (0, i, 0)),
        out_shape=jax.ShapeDtypeStruct((net, n, d), jnp.float32),
    )(x, conv_w)
    yf = y.reshape(net * n, d)

    gix = edge_attr * n + edge_index[0]
    if epad:
        pad0 = jnp.zeros((epad,), jnp.int32)
        gix = jnp.concatenate([gix, pad0]).reshape(_NW, nchunk, cb)
        dst = jnp.concatenate([edge_index[1], pad0 + n]).reshape(_NW, nchunk, cb)
    else:
        gix = gix.reshape(_NW, nchunk, cb)
        dst = edge_index[1].reshape(_NW, nchunk, cb)
    zeros = jnp.zeros((npad, d), jnp.float32)

    mesh = plsc.VectorSubcoreMesh(core_axis_name="c", subcore_axis_name="s",
                                  num_cores=_NC, num_subcores=_NS)

    @functools.partial(
        pl.kernel,
        out_type=jax.ShapeDtypeStruct((_NC, npad, d), jnp.float32),
        mesh=mesh,
        scratch_types=[
            pltpu.VMEM((nchunk, cb), jnp.int32),
            pltpu.VMEM((nchunk, cb), jnp.int32),
            pltpu.VMEM((cb, d), jnp.float32),
            pltpu.VMEM_SHARED((npad, d), jnp.float32),
            pltpu.SemaphoreType.DMA,
        ],
    )
    def _sc_edge(gix_h, dst_h, y_h, z_h, out_h,
                 gidx, didx, rb0, acc, sg0):
        cid = jax.lax.axis_index("c")
        sid = jax.lax.axis_index("s")
        wid = sid * _NC + cid
        pltpu.sync_copy(gix_h.at[wid], gidx)
        pltpu.sync_copy(dst_h.at[wid], didx)
        pltpu.sync_copy(z_h.at[pl.ds(sid * rpt, rpt)],
                        acc.at[pl.ds(sid * rpt, rpt)])
        plsc.subcore_barrier()

        def chunk(j, carry):
            pltpu.async_copy(y_h.at[gidx.at[j]], rb0, sg0).wait()
            pltpu.sync_copy(rb0, acc.at[didx.at[j]], add=True)
            return carry

        jax.lax.fori_loop(0, nchunk, chunk, 0)
        plsc.subcore_barrier()
        pltpu.sync_copy(acc.at[pl.ds(sid * rpt, rpt)],
                        out_h.at[cid, pl.ds(sid * rpt, rpt)])

    partials = _sc_edge(gix, dst, yf, zeros)

    btot = (lin_b + jnp.sum(conv_b, axis=0)).reshape(1, d)
    z = pl.pallas_call(
        _lin_body,
        grid=(nb,),
        in_specs=[pl.BlockSpec((bn, d), lambda i: (i, 0)),
                  pl.BlockSpec((d, d), lambda i: (0, 0)),
                  pl.BlockSpec((1, d), lambda i: (0, 0))],
        out_specs=pl.BlockSpec((bn, d), lambda i: (i, 0)),
        out_shape=jax.ShapeDtypeStruct((n, d), jnp.float32),
    )(x, lin_w, btot)

    out = pl.pallas_call(
        functools.partial(_bnfused_body, n, nb),
        grid=(2 * nb,),
        in_specs=[pl.BlockSpec((bn, d), lambda it: (jnp.where(it < nb, it, 0), 0)),
                  pl.BlockSpec((1, bn, d),
                               lambda it: (0, jnp.where(it < nb, it, 0), 0)),
                  pl.BlockSpec((1, bn, d),
                               lambda it: (1, jnp.where(it < nb, it, 0), 0)),
                  pl.BlockSpec((1, d), lambda it: (0, 0)),
                  pl.BlockSpec((1, d), lambda it: (0, 0))],
        out_specs=pl.BlockSpec((bn, d), lambda it: (it % nb, 0)),
        out_shape=jax.ShapeDtypeStruct((n, d), jnp.float32),
        scratch_shapes=[pltpu.VMEM((n, d), jnp.float32),
                        pltpu.VMEM((2, d), jnp.float32)],
    )(z, partials, partials, bn_gamma.reshape(1, d), bn_beta.reshape(1, d))
    return out


# speedup vs baseline: 1.0615x; 1.0037x over previous
import functools

import jax
import jax.numpy as jnp
from jax.experimental import pallas as pl
from jax.experimental.pallas import tpu as pltpu
from jax.experimental.pallas import tpu_sc as plsc

_NC = 2
_NS = 16
_NW = _NC * _NS


def _ymm_body(net, x_ref, w_ref, y_ref):
    xb = x_ref[...]
    for et in range(net):
        y_ref[et] = jax.lax.dot_general(
            xb, w_ref[et], (((1,), (1,)), ((), ())),
            preferred_element_type=jnp.float32)


def _lin_body(x_ref, w_ref, b_ref, z_ref):
    z_ref[...] = jax.lax.dot_general(
        x_ref[...], w_ref[...], (((1,), (1,)), ((), ())),
        preferred_element_type=jnp.float32) + b_ref[...]


def _bnfused_body(n_rows, nb, z_ref, p0_ref, p1_ref, g_ref, b_ref, o_ref,
                  h_ref, s_ref):
    it = pl.program_id(0)
    i = jax.lax.rem(it, nb)

    @pl.when(it < nb)
    def _():
        hb = jnp.maximum(z_ref[...] + p0_ref[0] + p1_ref[0], 0.0)
        h_ref[pl.ds(i * z_ref.shape[0], z_ref.shape[0]), :] = hb
        st = jnp.concatenate(
            [jnp.sum(hb, axis=0, keepdims=True),
             jnp.sum(hb * hb, axis=0, keepdims=True)], axis=0)

        @pl.when(it == 0)
        def _():
            s_ref[...] = st

        @pl.when(it > 0)
        def _():
            s_ref[...] = s_ref[...] + st

    @pl.when(it >= nb)
    def _():
        s = s_ref[...]
        mean = s[0:1] * (1.0 / n_rows)
        var = s[1:2] * (1.0 / n_rows) - mean * mean
        inv = jax.lax.rsqrt(var + 1e-5)
        hb = h_ref[pl.ds(i * o_ref.shape[0], o_ref.shape[0]), :]
        o_ref[...] = (hb - mean) * (g_ref[...] * inv) + b_ref[...]


def kernel(x, edge_index, edge_attr, conv_w, conv_b, lin_w, lin_b, bn_gamma, bn_beta):
    n, d = x.shape
    e = edge_attr.shape[0]
    net = conv_w.shape[0]

    cb = 80
    npad = ((n + _NS * 8 - 1) // (_NS * 8)) * (_NS * 8)
    rpt = npad // _NS
    nchunk = -(-e // (_NW * cb))
    epw = nchunk * cb
    epad = _NW * epw - e

    bn = 5000
    nb = n // bn
    y = pl.pallas_call(
        functools.partial(_ymm_body, net),
        grid=(nb,),
        in_specs=[pl.BlockSpec((bn, d), lambda i: (i, 0)),
                  pl.BlockSpec((net, d, d), lambda i: (0, 0, 0))],
        out_specs=pl.BlockSpec((net, bn, d), lambda i: (0, i, 0)),
        out_shape=jax.ShapeDtypeStruct((net, n, d), jnp.float32),
    )(x, conv_w)
    yf = y.reshape(net * n, d)

    gix = edge_attr * n + edge_index[0]
    if epad:
        pad0 = jnp.zeros((epad,), jnp.int32)
        gix = jnp.concatenate([gix, pad0]).reshape(_NW, nchunk, cb)
        dst = jnp.concatenate([edge_index[1], pad0 + n]).reshape(_NW, nchunk, cb)
    else:
        gix = gix.reshape(_NW, nchunk, cb)
        dst = edge_index[1].reshape(_NW, nchunk, cb)
    zeros = jnp.zeros((npad, d), jnp.float32)

    mesh = plsc.VectorSubcoreMesh(core_axis_name="c", subcore_axis_name="s",
                                  num_cores=_NC, num_subcores=_NS)

    @functools.partial(
        pl.kernel,
        out_type=jax.ShapeDtypeStruct((_NC, npad, d), jnp.float32),
        mesh=mesh,
        scratch_types=[
            pltpu.VMEM((nchunk, cb), jnp.int32),
            pltpu.VMEM((nchunk, cb), jnp.int32),
            pltpu.VMEM((cb, d), jnp.float32),
            pltpu.VMEM_SHARED((npad, d), jnp.float32),
            pltpu.SemaphoreType.DMA,
        ],
    )
    def _sc_edge(gix_h, dst_h, y_h, z_h, out_h,
                 gidx, didx, rb0, acc, sg0):
        cid = jax.lax.axis_index("c")
        sid = jax.lax.axis_index("s")
        wid = sid * _NC + cid
        pltpu.sync_copy(gix_h.at[wid], gidx)
        pltpu.sync_copy(dst_h.at[wid], didx)
        pltpu.sync_copy(z_h.at[pl.ds(sid * rpt, rpt)],
                        acc.at[pl.ds(sid * rpt, rpt)])
        plsc.subcore_barrier()

        def chunk(j, carry):
            pltpu.async_copy(y_h.at[gidx.at[j]], rb0, sg0).wait()
            pltpu.sync_copy(rb0, acc.at[didx.at[j]], add=True)
            return carry

        jax.lax.fori_loop(0, nchunk, chunk, 0)
        plsc.subcore_barrier()
        pltpu.sync_copy(acc.at[pl.ds(sid * rpt, rpt)],
                        out_h.at[cid, pl.ds(sid * rpt, rpt)])

    partials = _sc_edge(gix, dst, yf, zeros)

    btot = (lin_b + jnp.sum(conv_b, axis=0)).reshape(1, d)
    z = pl.pallas_call(
        _lin_body,
        grid=(nb,),
        in_specs=[pl.BlockSpec((bn, d), lambda i: (i, 0)),
                  pl.BlockSpec((d, d), lambda i: (0, 0)),
                  pl.BlockSpec((1, d), lambda i: (0, 0))],
        out_specs=pl.BlockSpec((bn, d), lambda i: (i, 0)),
        out_shape=jax.ShapeDtypeStruct((n, d), jnp.float32),
    )(x, lin_w, btot)

    out = pl.pallas_call(
        functools.partial(_bnfused_body, n, nb),
        grid=(2 * nb,),
        in_specs=[pl.BlockSpec((bn, d), lambda it: (jnp.where(it < nb, it, 0), 0)),
                  pl.BlockSpec((1, bn, d),
                               lambda it: (0, jnp.where(it < nb, it, 0), 0)),
                  pl.BlockSpec((1, bn, d),
                               lambda it: (1, jnp.where(it < nb, it, 0), 0)),
                  pl.BlockSpec((1, d), lambda it: (0, 0)),
                  pl.BlockSpec((1, d), lambda it: (0, 0))],
        out_specs=pl.BlockSpec((bn, d), lambda it: (it % nb, 0)),
        out_shape=jax.ShapeDtypeStruct((n, d), jnp.float32),
        scratch_shapes=[pltpu.VMEM((n, d), jnp.float32),
                        pltpu.VMEM((2, d), jnp.float32)],
    )(z, partials, partials, bn_gamma.reshape(1, d), bn_beta.reshape(1, d))
    return out
